# baseline (speedup 1.0000x reference)
import functools

import jax
import jax.numpy as jnp
from jax import lax
from jax.experimental import pallas as pl
from jax.experimental.pallas import tpu as pltpu
from jax.experimental.pallas import tpu_sc as plsc

N_NODES = 10000
N_TARGET = 8192
E = 160000
L = 3
D = 128
H = 4
AV = 128
OUT_DIM = 128

W = 256
NB = N_TARGET // W
C = 512
E_PAD = E + C

NC = 2
NS = 16
NW = NC * NS
EDGES_PER_W = 2 * E // NW
EK = 80
ECH = EDGES_PER_W // EK
CEN_PER_W = 2 * N_TARGET // NW
CK = 256
A1P = 128


def _sc_gather_body(feat, topic, idxT, txt, nlst, fa1,
                    hid_out, a1_out,
                    iea0, iea1, iea2, iea3, ieb0, ieb1, ieb2, ieb3,
                    ra0, ra1, ra2, ra3, rb0, rb1, rb2, rb3,
                    icen, rcen, sema, semb, semc):
    wid = lax.axis_index("s") * NC + lax.axis_index("c")
    mp = wid // NS
    ww = wid % NS
    ebase = ww * EDGES_PER_W
    mpE = mp * (L * E)
    mpT = mp * E

    def fire(k, ie0, ie1, ie2, ie3, r0, r1, r2, r3, sem):
        b = ebase + k * EK
        pltpu.sync_copy(idxT.at[pl.ds(mpE + b, EK)], ie0)
        pltpu.sync_copy(idxT.at[pl.ds(mpE + E + b, EK)], ie1)
        pltpu.sync_copy(idxT.at[pl.ds(mpE + 2 * E + b, EK)], ie2)
        pltpu.sync_copy(txt.at[pl.ds(mpT + b, EK)], ie3)
        pltpu.async_copy(feat.at[ie0], r0, sem)
        pltpu.async_copy(feat.at[ie1], r1, sem)
        pltpu.async_copy(feat.at[ie2], r2, sem)
        pltpu.async_copy(topic.at[ie3], r3, sem)

    def drain(ie0, ie1, ie2, ie3, r0, r1, r2, r3, sem):
        pltpu.make_async_copy(feat.at[ie0], r0, sem).wait()
        pltpu.make_async_copy(feat.at[ie1], r1, sem).wait()
        pltpu.make_async_copy(feat.at[ie2], r2, sem).wait()
        pltpu.make_async_copy(topic.at[ie3], r3, sem).wait()

    def compute_store(k, r0, r1, r2, r3):
        def erow(e, c2):
            for u in range(2):
                ee = e * 2 + u
                for d8 in range(D // 16):
                    s = pl.ds(d8 * 16, 16)
                    r0[ee, s] = (r0[ee, s] + r1[ee, s] + r2[ee, s]) \
                        * (1.0 / 3.0) + r3[ee, s]
            return c2
        lax.fori_loop(0, EK // 2, erow, 0)
        pltpu.sync_copy(r0, hid_out.at[mp, pl.ds(ebase + k * EK, EK)])

    bufs_a = (iea0, iea1, iea2, iea3, ra0, ra1, ra2, ra3, sema)
    bufs_b = (ieb0, ieb1, ieb2, ieb3, rb0, rb1, rb2, rb3, semb)

    def body(k, carry):
        fire(k, *bufs_a)
        drain(*bufs_a)
        compute_store(k, ra0, ra1, ra2, ra3)
        return carry
    lax.fori_loop(0, ECH, body, 0)
    del bufs_b

    cbase = ww * CEN_PER_W

    def cchunk(k, carry):
        b = cbase + k * CK
        pltpu.sync_copy(nlst.at[pl.ds(mp * N_TARGET + b, CK)], icen)
        pltpu.async_copy(fa1.at[icen], rcen, semc).wait()
        pltpu.sync_copy(rcen, a1_out.at[mp, pl.ds(b, CK)])
        return carry
    lax.fori_loop(0, CEN_PER_W // CK, cchunk, 0)


def _sc_gather(feat, topic, idxT, txt, nlst, fa1):
    mesh = plsc.VectorSubcoreMesh(core_axis_name="c", subcore_axis_name="s")
    fn = pl.kernel(
        _sc_gather_body,
        mesh=mesh,
        out_type=(
            jax.ShapeDtypeStruct((2, E_PAD, D), jnp.float32),
            jax.ShapeDtypeStruct((2, N_TARGET, A1P), jnp.float32),
        ),
        scratch_types=(
            [pltpu.VMEM((EK,), jnp.int32) for _ in range(8)]
            + [pltpu.VMEM((EK, D), jnp.float32) for _ in range(8)]
            + [pltpu.VMEM((CK,), jnp.int32),
               pltpu.VMEM((CK, A1P), jnp.float32),
               pltpu.SemaphoreType.DMA,
               pltpu.SemaphoreType.DMA,
               pltpu.SemaphoreType.DMA]
        ),
    )
    return fn(feat, topic, idxT, txt, nlst, fa1)


def _fa1_body(feat_ref, attn1p_ref, out_ref):
    out_ref[...] = jnp.dot(feat_ref[...], attn1p_ref[...],
                           preferred_element_type=jnp.float32)


def _tc_fa1(features, attn1p):
    return pl.pallas_call(
        _fa1_body,
        out_shape=jax.ShapeDtypeStruct((N_NODES, A1P), jnp.float32),
    )(features, attn1p)


def _tcb_body(off_ref, hid_hbm, tgt_hbm, a1_ref, attn2_ref,
              fc1w_ref, fc1b_ref, fc2w_ref,
              h_ref, s_ref, hid_buf, tgt_buf, accn, accd, sem1, sem2):
    m = pl.program_id(0)
    t = pl.program_id(1)
    start = off_ref[m, t]
    end = off_ref[m, t + 1]
    astart = (start // C) * C
    trip = (end - astart + C - 1) // C

    accn[...] = jnp.zeros((W, H * D), jnp.float32)
    accd[...] = jnp.zeros((W, H), jnp.float32)

    def chunk(c, carry):
        s0 = astart + c * C
        cp1 = pltpu.make_async_copy(hid_hbm.at[m, pl.ds(s0, C), :],
                                    hid_buf, sem1)
        cp2 = pltpu.make_async_copy(tgt_hbm.at[m, pl.ds(s0, C), :],
                                    tgt_buf, sem2)
        cp1.start()
        cp2.start()
        cp1.wait()
        cp2.wait()
        tv = tgt_buf[...]
        jg = lax.broadcasted_iota(jnp.int32, (C, 1), 0) + s0
        mask = (jg >= start) & (jg < end)
        rel = jnp.clip(tv - t * W, 0, W - 1)
        onehot = jnp.where(
            (rel == lax.broadcasted_iota(jnp.int32, (C, W), 1)) & mask,
            1.0, 0.0)
        hidm = jnp.where(mask, hid_buf[...], 0.0)
        a2 = lax.dot_general(hidm, attn2_ref[...],
                             (((1,), (1,)), ((), ())),
                             preferred_element_type=jnp.float32)
        a1e = jnp.dot(onehot, a1_ref[0][:, :H],
                      preferred_element_type=jnp.float32)
        a = a1e + a2
        a = jnp.where(a > 0, a, 0.01 * a)
        ae = jnp.where(mask, jnp.exp(a), 0.0)
        vals = jnp.concatenate(
            [ae[:, h:h + 1] * hidm for h in range(H)], axis=1)
        accn[...] += lax.dot_general(onehot, vals,
                                     (((0,), (0,)), ((), ())),
                                     preferred_element_type=jnp.float32)
        accd[...] += lax.dot_general(onehot, ae,
                                     (((0,), (0,)), ((), ())),
                                     preferred_element_type=jnp.float32)
        return carry
    lax.fori_loop(0, trip, chunk, 0)

    num = accn[...]
    den = accd[...]
    hp = jnp.concatenate(
        [num[:, h * D:(h + 1) * D] / (den[:, h:h + 1] + 1e-9)
         for h in range(H)], axis=1)
    hp = jnp.where(hp > 0, hp, jnp.exp(jnp.minimum(hp, 0.0)) - 1.0)
    h_ref[0] = hp

    q = jnp.tanh(jnp.dot(hp, fc1w_ref[...],
                         preferred_element_type=jnp.float32)
                 + fc1b_ref[...][None, :])
    sp = jnp.dot(q, fc2w_ref[...], preferred_element_type=jnp.float32)
    ssum = jnp.sum(sp)

    first = (m == 0) & (t == 0)

    @pl.when(first)
    def _init():
        s_ref[...] = jnp.zeros((2, 1), jnp.float32)

    sel = lax.broadcasted_iota(jnp.int32, (2, 1), 0) == m
    s_ref[...] += jnp.where(sel, ssum, 0.0)


def _tc_b(off, hidden, tgt3, a1, attn2, fc1_w, fc1_b, fc2_w):
    return pl.pallas_call(
        _tcb_body,
        grid=(2, NB),
        in_specs=[
            pl.BlockSpec(memory_space=pltpu.SMEM),
            pl.BlockSpec(memory_space=pl.ANY),
            pl.BlockSpec(memory_space=pl.ANY),
            pl.BlockSpec((1, W, A1P), lambda m, t: (m, t, 0)),
            pl.BlockSpec((H, D), lambda m, t: (0, 0)),
            pl.BlockSpec((H * D, AV), lambda m, t: (0, 0)),
            pl.BlockSpec((AV,), lambda m, t: (0,)),
            pl.BlockSpec((AV, 1), lambda m, t: (0, 0)),
        ],
        out_specs=[
            pl.BlockSpec((1, W, H * D), lambda m, t: (m, t, 0)),
            pl.BlockSpec((2, 1), lambda m, t: (0, 0)),
        ],
        out_shape=[
            jax.ShapeDtypeStruct((2, N_TARGET, H * D), jnp.float32),
            jax.ShapeDtypeStruct((2, 1), jnp.float32),
        ],
        scratch_shapes=[
            pltpu.VMEM((C, D), jnp.float32),
            pltpu.VMEM((C, 1), jnp.int32),
            pltpu.VMEM((W, H * D), jnp.float32),
            pltpu.VMEM((W, H), jnp.float32),
            pltpu.SemaphoreType.DMA,
            pltpu.SemaphoreType.DMA,
        ],
    )(off, hidden, tgt3, a1, attn2, fc1_w, fc1_b, fc2_w)


def _tcc_body(h_ref, s_ref, fcuw_ref, fcub_ref,
              hu_ref, lg_ref, beta_ref):
    sv = s_ref[...] * (1.0 / N_TARGET)
    ex = jnp.exp(sv - jnp.max(sv))
    beta = ex / jnp.sum(ex)

    t = pl.program_id(0)

    @pl.when(t == 0)
    def _():
        beta_ref[...] = beta

    hu = beta[0:1, 0:1] * h_ref[0] + beta[1:2, 0:1] * h_ref[1]
    hu_ref[...] = hu
    lg_ref[...] = jnp.dot(hu, fcuw_ref[...],
                          preferred_element_type=jnp.float32) \
        + fcub_ref[...][None, :]


def _tc_c(h_all, s_all, fc_user_w, fc_user_b):
    return pl.pallas_call(
        _tcc_body,
        grid=(NB,),
        in_specs=[
            pl.BlockSpec((2, W, H * D), lambda t: (0, t, 0)),
            pl.BlockSpec((2, 1), lambda t: (0, 0)),
            pl.BlockSpec((H * D, OUT_DIM), lambda t: (0, 0)),
            pl.BlockSpec((OUT_DIM,), lambda t: (0,)),
        ],
        out_specs=[
            pl.BlockSpec((W, H * D), lambda t: (t, 0)),
            pl.BlockSpec((W, OUT_DIM), lambda t: (t, 0)),
            pl.BlockSpec((2, 1), lambda t: (0, 0)),
        ],
        out_shape=[
            jax.ShapeDtypeStruct((N_TARGET, H * D), jnp.float32),
            jax.ShapeDtypeStruct((N_TARGET, OUT_DIM), jnp.float32),
            jax.ShapeDtypeStruct((2, 1), jnp.float32),
        ],
    )(h_all, s_all, fc_user_w, fc_user_b)


def kernel(features, topic, type_mask,
           edge_metapath_indices_0, edge_metapath_indices_1,
           edge_metapath_text_indices_0, edge_metapath_text_indices_1,
           target_idx_0, target_idx_1, node_list_0, node_list_1,
           attn1, attn2, fc1_w, fc1_b, fc2_w, fc_user_w, fc_user_b):
    del type_mask
    idxT = jnp.stack([edge_metapath_indices_0.T,
                      edge_metapath_indices_1.T]).astype(jnp.int32).reshape(-1)
    txts = jnp.stack([edge_metapath_text_indices_0,
                      edge_metapath_text_indices_1]).astype(jnp.int32).reshape(-1)
    nls = jnp.stack([node_list_0, node_list_1]).astype(jnp.int32).reshape(-1)
    zpad = jnp.zeros((C,), jnp.int32)
    tgt3 = jnp.stack([
        jnp.concatenate([target_idx_0.astype(jnp.int32), zpad]),
        jnp.concatenate([target_idx_1.astype(jnp.int32), zpad]),
    ]).reshape(2, E_PAD, 1)
    bnd = jnp.arange(NB + 1, dtype=jnp.int32) * W
    off = jnp.stack([
        jnp.searchsorted(target_idx_0, bnd),
        jnp.searchsorted(target_idx_1, bnd),
    ]).astype(jnp.int32)

    attn1p = jnp.pad(attn1, ((0, 0), (0, A1P - H)))
    fa1 = _tc_fa1(features, attn1p)
    hidden, a1 = _sc_gather(features, topic, idxT, txts, nls, fa1)
    h_all, s_all = _tc_b(off, hidden, tgt3, a1, attn2, fc1_w, fc1_b, fc2_w)
    h_user, logits, beta2 = _tc_c(h_all, s_all, fc_user_w, fc_user_b)
    return h_user, logits, beta2.reshape(2)


# speedup vs baseline: 1.2946x; 1.2946x over previous
import functools

import jax
import jax.numpy as jnp
from jax import lax
from jax.experimental import pallas as pl
from jax.experimental.pallas import tpu as pltpu
from jax.experimental.pallas import tpu_sc as plsc

N_NODES = 10000
N_TARGET = 8192
E = 160000
L = 3
D = 128
H = 4
AV = 128
OUT_DIM = 128

W = 256
NB = N_TARGET // W
C = 1024
E_PAD = E + C

NC = 2
NS = 16
NW = NC * NS
EDGES_PER_W = 2 * E // NW
EK = 200
ECH = EDGES_PER_W // EK
CEN_PER_W = 2 * N_TARGET // NW
CK = 128
A1P = 128


def _sc_gather_body(feat, topic, idxT, txt, nlst, fa1,
                    hid_out, a1_out,
                    iea0, iea1, iea2, iea3,
                    ra0, ra1, ra2, ra3,
                    icen, rcen, sema, semc):
    wid = lax.axis_index("s") * NC + lax.axis_index("c")
    mp = wid // NS
    ww = wid % NS
    ebase = ww * EDGES_PER_W
    mpE = mp * (L * E)
    mpT = mp * E

    def fire(k, ie0, ie1, ie2, ie3, r0, r1, r2, r3, sem):
        b = ebase + k * EK
        pltpu.sync_copy(idxT.at[pl.ds(mpE + b, EK)], ie0)
        pltpu.sync_copy(idxT.at[pl.ds(mpE + E + b, EK)], ie1)
        pltpu.sync_copy(idxT.at[pl.ds(mpE + 2 * E + b, EK)], ie2)
        pltpu.sync_copy(txt.at[pl.ds(mpT + b, EK)], ie3)
        pltpu.async_copy(feat.at[ie0], r0, sem)
        pltpu.async_copy(feat.at[ie1], r1, sem)
        pltpu.async_copy(feat.at[ie2], r2, sem)
        pltpu.async_copy(topic.at[ie3], r3, sem)

    def drain(ie0, ie1, ie2, ie3, r0, r1, r2, r3, sem):
        pltpu.make_async_copy(feat.at[ie0], r0, sem).wait()
        pltpu.make_async_copy(feat.at[ie1], r1, sem).wait()
        pltpu.make_async_copy(feat.at[ie2], r2, sem).wait()
        pltpu.make_async_copy(topic.at[ie3], r3, sem).wait()

    def compute_store(k, r0, r1, r2, r3):
        def erow(e, c2):
            for u in range(2):
                ee = e * 2 + u
                for d8 in range(D // 16):
                    s = pl.ds(d8 * 16, 16)
                    r0[ee, s] = (r0[ee, s] + r1[ee, s] + r2[ee, s]) \
                        * (1.0 / 3.0) + r3[ee, s]
            return c2
        lax.fori_loop(0, EK // 2, erow, 0)
        pltpu.sync_copy(r0, hid_out.at[mp, pl.ds(ebase + k * EK, EK)])

    bufs_a = (iea0, iea1, iea2, iea3, ra0, ra1, ra2, ra3, sema)

    def body(k, carry):
        fire(k, *bufs_a)
        drain(*bufs_a)
        compute_store(k, ra0, ra1, ra2, ra3)
        return carry
    lax.fori_loop(0, ECH, body, 0)

    cbase = ww * CEN_PER_W

    def cchunk(k, carry):
        b = cbase + k * CK
        pltpu.sync_copy(nlst.at[pl.ds(mp * N_TARGET + b, CK)], icen)
        pltpu.async_copy(fa1.at[icen], rcen, semc).wait()
        pltpu.sync_copy(rcen, a1_out.at[mp, pl.ds(b, CK)])
        return carry
    lax.fori_loop(0, CEN_PER_W // CK, cchunk, 0)


def _sc_gather(feat, topic, idxT, txt, nlst, fa1):
    mesh = plsc.VectorSubcoreMesh(core_axis_name="c", subcore_axis_name="s")
    fn = pl.kernel(
        _sc_gather_body,
        mesh=mesh,
        out_type=(
            jax.ShapeDtypeStruct((2, E_PAD, D), jnp.float32),
            jax.ShapeDtypeStruct((2, N_TARGET, A1P), jnp.float32),
        ),
        scratch_types=(
            [pltpu.VMEM((EK,), jnp.int32) for _ in range(4)]
            + [pltpu.VMEM((EK, D), jnp.float32) for _ in range(4)]
            + [pltpu.VMEM((CK,), jnp.int32),
               pltpu.VMEM((CK, A1P), jnp.float32),
               pltpu.SemaphoreType.DMA,
               pltpu.SemaphoreType.DMA]
        ),
    )
    return fn(feat, topic, idxT, txt, nlst, fa1)


def _fa1_body(feat_ref, attn1p_ref, out_ref):
    out_ref[...] = jnp.dot(feat_ref[...], attn1p_ref[...],
                           preferred_element_type=jnp.float32)


def _tc_fa1(features, attn1p):
    return pl.pallas_call(
        _fa1_body,
        out_shape=jax.ShapeDtypeStruct((N_NODES, A1P), jnp.float32),
    )(features, attn1p)


def _tcb_body(off_ref, hid_hbm, tgt_hbm, a1_ref, attn2_ref,
              fc1w_ref, fc1b_ref, fc2w_ref,
              h_ref, s_ref, hid_buf, tgt_buf, accn, accd, sem1, sem2):
    m = pl.program_id(0)
    t = pl.program_id(1)
    start = off_ref[m, t]
    end = off_ref[m, t + 1]
    astart = (start // C) * C
    trip = (end - astart + C - 1) // C

    accn[...] = jnp.zeros((W, H * D), jnp.float32)
    accd[...] = jnp.zeros((W, H), jnp.float32)

    def chunk(c, carry):
        s0 = astart + c * C
        cp1 = pltpu.make_async_copy(hid_hbm.at[m, pl.ds(s0, C), :],
                                    hid_buf, sem1)
        cp2 = pltpu.make_async_copy(tgt_hbm.at[m, pl.ds(s0, C), :],
                                    tgt_buf, sem2)
        cp1.start()
        cp2.start()
        cp1.wait()
        cp2.wait()
        tv = tgt_buf[...]
        jg = lax.broadcasted_iota(jnp.int32, (C, 1), 0) + s0
        mask = (jg >= start) & (jg < end)
        rel = jnp.clip(tv - t * W, 0, W - 1)
        onehot = jnp.where(
            (rel == lax.broadcasted_iota(jnp.int32, (C, W), 1)) & mask,
            1.0, 0.0)
        hidm = jnp.where(mask, hid_buf[...], 0.0)
        a2 = lax.dot_general(hidm, attn2_ref[...],
                             (((1,), (1,)), ((), ())),
                             preferred_element_type=jnp.float32)
        a1e = jnp.dot(onehot, a1_ref[0][:, :H],
                      preferred_element_type=jnp.float32)
        a = a1e + a2
        a = jnp.where(a > 0, a, 0.01 * a)
        ae = jnp.where(mask, jnp.exp(a), 0.0)
        vals = jnp.concatenate(
            [ae[:, h:h + 1] * hidm for h in range(H)], axis=1)
        accn[...] += lax.dot_general(onehot, vals,
                                     (((0,), (0,)), ((), ())),
                                     preferred_element_type=jnp.float32)
        accd[...] += lax.dot_general(onehot, ae,
                                     (((0,), (0,)), ((), ())),
                                     preferred_element_type=jnp.float32)
        return carry
    lax.fori_loop(0, trip, chunk, 0)

    num = accn[...]
    den = accd[...]
    hp = jnp.concatenate(
        [num[:, h * D:(h + 1) * D] / (den[:, h:h + 1] + 1e-9)
         for h in range(H)], axis=1)
    hp = jnp.where(hp > 0, hp, jnp.exp(jnp.minimum(hp, 0.0)) - 1.0)
    h_ref[0] = hp

    q = jnp.tanh(jnp.dot(hp, fc1w_ref[...],
                         preferred_element_type=jnp.float32)
                 + fc1b_ref[...][None, :])
    sp = jnp.dot(q, fc2w_ref[...], preferred_element_type=jnp.float32)
    ssum = jnp.sum(sp)

    first = (m == 0) & (t == 0)

    @pl.when(first)
    def _init():
        s_ref[...] = jnp.zeros((2, 1), jnp.float32)

    sel = lax.broadcasted_iota(jnp.int32, (2, 1), 0) == m
    s_ref[...] += jnp.where(sel, ssum, 0.0)


def _tc_b(off, hidden, tgt3, a1, attn2, fc1_w, fc1_b, fc2_w):
    return pl.pallas_call(
        _tcb_body,
        grid=(2, NB),
        in_specs=[
            pl.BlockSpec(memory_space=pltpu.SMEM),
            pl.BlockSpec(memory_space=pl.ANY),
            pl.BlockSpec(memory_space=pl.ANY),
            pl.BlockSpec((1, W, A1P), lambda m, t: (m, t, 0)),
            pl.BlockSpec((H, D), lambda m, t: (0, 0)),
            pl.BlockSpec((H * D, AV), lambda m, t: (0, 0)),
            pl.BlockSpec((AV,), lambda m, t: (0,)),
            pl.BlockSpec((AV, 1), lambda m, t: (0, 0)),
        ],
        out_specs=[
            pl.BlockSpec((1, W, H * D), lambda m, t: (m, t, 0)),
            pl.BlockSpec((2, 1), lambda m, t: (0, 0)),
        ],
        out_shape=[
            jax.ShapeDtypeStruct((2, N_TARGET, H * D), jnp.float32),
            jax.ShapeDtypeStruct((2, 1), jnp.float32),
        ],
        scratch_shapes=[
            pltpu.VMEM((C, D), jnp.float32),
            pltpu.VMEM((C, 1), jnp.int32),
            pltpu.VMEM((W, H * D), jnp.float32),
            pltpu.VMEM((W, H), jnp.float32),
            pltpu.SemaphoreType.DMA,
            pltpu.SemaphoreType.DMA,
        ],
    )(off, hidden, tgt3, a1, attn2, fc1_w, fc1_b, fc2_w)


def _tcc_body(h_ref, s_ref, fcuw_ref, fcub_ref,
              hu_ref, lg_ref, beta_ref):
    sv = s_ref[...] * (1.0 / N_TARGET)
    ex = jnp.exp(sv - jnp.max(sv))
    beta = ex / jnp.sum(ex)

    t = pl.program_id(0)

    @pl.when(t == 0)
    def _():
        beta_ref[...] = beta

    hu = beta[0:1, 0:1] * h_ref[0] + beta[1:2, 0:1] * h_ref[1]
    hu_ref[...] = hu
    lg_ref[...] = jnp.dot(hu, fcuw_ref[...],
                          preferred_element_type=jnp.float32) \
        + fcub_ref[...][None, :]


def _tc_c(h_all, s_all, fc_user_w, fc_user_b):
    return pl.pallas_call(
        _tcc_body,
        grid=(NB,),
        in_specs=[
            pl.BlockSpec((2, W, H * D), lambda t: (0, t, 0)),
            pl.BlockSpec((2, 1), lambda t: (0, 0)),
            pl.BlockSpec((H * D, OUT_DIM), lambda t: (0, 0)),
            pl.BlockSpec((OUT_DIM,), lambda t: (0,)),
        ],
        out_specs=[
            pl.BlockSpec((W, H * D), lambda t: (t, 0)),
            pl.BlockSpec((W, OUT_DIM), lambda t: (t, 0)),
            pl.BlockSpec((2, 1), lambda t: (0, 0)),
        ],
        out_shape=[
            jax.ShapeDtypeStruct((N_TARGET, H * D), jnp.float32),
            jax.ShapeDtypeStruct((N_TARGET, OUT_DIM), jnp.float32),
            jax.ShapeDtypeStruct((2, 1), jnp.float32),
        ],
    )(h_all, s_all, fc_user_w, fc_user_b)


def kernel(features, topic, type_mask,
           edge_metapath_indices_0, edge_metapath_indices_1,
           edge_metapath_text_indices_0, edge_metapath_text_indices_1,
           target_idx_0, target_idx_1, node_list_0, node_list_1,
           attn1, attn2, fc1_w, fc1_b, fc2_w, fc_user_w, fc_user_b):
    del type_mask
    idxT = jnp.stack([edge_metapath_indices_0.T,
                      edge_metapath_indices_1.T]).astype(jnp.int32).reshape(-1)
    txts = jnp.stack([edge_metapath_text_indices_0,
                      edge_metapath_text_indices_1]).astype(jnp.int32).reshape(-1)
    nls = jnp.stack([node_list_0, node_list_1]).astype(jnp.int32).reshape(-1)
    zpad = jnp.zeros((C,), jnp.int32)
    tgt3 = jnp.stack([
        jnp.concatenate([target_idx_0.astype(jnp.int32), zpad]),
        jnp.concatenate([target_idx_1.astype(jnp.int32), zpad]),
    ]).reshape(2, E_PAD, 1)
    bnd = jnp.arange(NB + 1, dtype=jnp.int32) * W
    off = jnp.stack([
        jnp.searchsorted(target_idx_0, bnd),
        jnp.searchsorted(target_idx_1, bnd),
    ]).astype(jnp.int32)

    attn1p = jnp.pad(attn1, ((0, 0), (0, A1P - H)))
    fa1 = _tc_fa1(features, attn1p)
    hidden, a1 = _sc_gather(features, topic, idxT, txts, nls, fa1)
    h_all, s_all = _tc_b(off, hidden, tgt3, a1, attn2, fc1_w, fc1_b, fc2_w)
    h_user, logits, beta2 = _tc_c(h_all, s_all, fc_user_w, fc_user_b)
    return h_user, logits, beta2.reshape(2)


# speedup vs baseline: 1.5314x; 1.1829x over previous
import jax
import jax.numpy as jnp
from jax import lax
from jax.experimental import pallas as pl
from jax.experimental.pallas import tpu as pltpu
from jax.experimental.pallas import tpu_sc as plsc

N_NODES = 10000
N_TARGET = 8192
E = 160000
L = 3
D = 128
H = 4
AV = 128
OUT_DIM = 128

W = 256
NB = N_TARGET // W
C = 1024
E_PAD = E + C

NC = 2
NS = 16
NW = NC * NS
EDGES_PER_W = E // NW
EK = 200
ECH = EDGES_PER_W // EK
CEN_PER_W = N_TARGET // NW
CK = 128
A1P = 128


def _sc_gather_body(feat, topic, idxT, txt, nlst, fa1,
                    hid_out, a1_out,
                    ie0, ie1, ie2, ie3, r0, r1, r2, r3,
                    icen, rcen, sema, semc):
    wid = lax.axis_index("s") * NC + lax.axis_index("c")
    ebase = wid * EDGES_PER_W

    def fire(k):
        b = ebase + k * EK
        pltpu.sync_copy(idxT.at[pl.ds(b, EK)], ie0)
        pltpu.sync_copy(idxT.at[pl.ds(E + b, EK)], ie1)
        pltpu.sync_copy(idxT.at[pl.ds(2 * E + b, EK)], ie2)
        pltpu.sync_copy(txt.at[pl.ds(b, EK)], ie3)
        pltpu.async_copy(feat.at[ie0], r0, sema)
        pltpu.async_copy(feat.at[ie1], r1, sema)
        pltpu.async_copy(feat.at[ie2], r2, sema)
        pltpu.async_copy(topic.at[ie3], r3, sema)

    def drain():
        pltpu.make_async_copy(feat.at[ie0], r0, sema).wait()
        pltpu.make_async_copy(feat.at[ie1], r1, sema).wait()
        pltpu.make_async_copy(feat.at[ie2], r2, sema).wait()
        pltpu.make_async_copy(topic.at[ie3], r3, sema).wait()

    def compute_store(k):
        def erow(e, c2):
            for u in range(2):
                ee = e * 2 + u
                for d8 in range(D // 16):
                    s = pl.ds(d8 * 16, 16)
                    r0[ee, s] = (r0[ee, s] + r1[ee, s] + r2[ee, s]) \
                        * (1.0 / 3.0) + r3[ee, s]
            return c2
        lax.fori_loop(0, EK // 2, erow, 0)
        pltpu.sync_copy(r0, hid_out.at[pl.ds(ebase + k * EK, EK)])

    def body(k, carry):
        fire(k)
        drain()
        compute_store(k)
        return carry
    lax.fori_loop(0, ECH, body, 0)

    cbase = wid * CEN_PER_W

    def cchunk(k, carry):
        b = cbase + k * CK
        pltpu.sync_copy(nlst.at[pl.ds(b, CK)], icen)
        pltpu.async_copy(fa1.at[icen], rcen, semc).wait()
        pltpu.sync_copy(rcen, a1_out.at[pl.ds(b, CK)])
        return carry
    lax.fori_loop(0, CEN_PER_W // CK, cchunk, 0)


def _sc_gather(feat, topic, idxT, txt, nlst, fa1):
    mesh = plsc.VectorSubcoreMesh(core_axis_name="c", subcore_axis_name="s")
    fn = pl.kernel(
        _sc_gather_body,
        mesh=mesh,
        out_type=(
            jax.ShapeDtypeStruct((E_PAD, D), jnp.float32),
            jax.ShapeDtypeStruct((N_TARGET, A1P), jnp.float32),
        ),
        scratch_types=(
            [pltpu.VMEM((EK,), jnp.int32) for _ in range(4)]
            + [pltpu.VMEM((EK, D), jnp.float32) for _ in range(4)]
            + [pltpu.VMEM((CK,), jnp.int32),
               pltpu.VMEM((CK, A1P), jnp.float32),
               pltpu.SemaphoreType.DMA,
               pltpu.SemaphoreType.DMA]
        ),
    )
    return fn(feat, topic, idxT, txt, nlst, fa1)


def _fa1_body(feat_ref, attn1p_ref, out_ref):
    out_ref[...] = jnp.dot(feat_ref[...], attn1p_ref[...],
                           preferred_element_type=jnp.float32)


def _tc_fa1(features, attn1p):
    return pl.pallas_call(
        _fa1_body,
        out_shape=jax.ShapeDtypeStruct((N_NODES, A1P), jnp.float32),
    )(features, attn1p)


def _tcb_body(off_ref, hid_hbm, tgt_hbm, a1_ref, attn2_ref,
              fc1w_ref, fc1b_ref, fc2w_ref,
              h_ref, s_ref, hid_buf, tgt_buf, accn, accd, sem1, sem2):
    t = pl.program_id(0)
    start = off_ref[t]
    end = off_ref[t + 1]
    astart = (start // C) * C
    trip = (end - astart + C - 1) // C

    accn[...] = jnp.zeros((W, H * D), jnp.float32)
    accd[...] = jnp.zeros((W, H), jnp.float32)

    def chunk(c, carry):
        s0 = astart + c * C
        cp1 = pltpu.make_async_copy(hid_hbm.at[pl.ds(s0, C), :],
                                    hid_buf, sem1)
        cp2 = pltpu.make_async_copy(tgt_hbm.at[pl.ds(s0, C), :],
                                    tgt_buf, sem2)
        cp1.start()
        cp2.start()
        cp1.wait()
        cp2.wait()
        tv = tgt_buf[...]
        jg = lax.broadcasted_iota(jnp.int32, (C, 1), 0) + s0
        mask = (jg >= start) & (jg < end)
        rel = jnp.clip(tv - t * W, 0, W - 1)
        onehot = jnp.where(
            (rel == lax.broadcasted_iota(jnp.int32, (C, W), 1)) & mask,
            1.0, 0.0)
        hidm = jnp.where(mask, hid_buf[...], 0.0)
        a2 = lax.dot_general(hidm, attn2_ref[...],
                             (((1,), (1,)), ((), ())),
                             preferred_element_type=jnp.float32)
        a1e = jnp.dot(onehot, a1_ref[:, :H],
                      preferred_element_type=jnp.float32)
        a = a1e + a2
        a = jnp.where(a > 0, a, 0.01 * a)
        ae = jnp.where(mask, jnp.exp(a), 0.0)
        vals = jnp.concatenate(
            [ae[:, h:h + 1] * hidm for h in range(H)], axis=1)
        accn[...] += lax.dot_general(onehot, vals,
                                     (((0,), (0,)), ((), ())),
                                     preferred_element_type=jnp.float32)
        accd[...] += lax.dot_general(onehot, ae,
                                     (((0,), (0,)), ((), ())),
                                     preferred_element_type=jnp.float32)
        return carry
    lax.fori_loop(0, trip, chunk, 0)

    num = accn[...]
    den = accd[...]
    hp = jnp.concatenate(
        [num[:, h * D:(h + 1) * D] / (den[:, h:h + 1] + 1e-9)
         for h in range(H)], axis=1)
    hp = jnp.where(hp > 0, hp, jnp.exp(jnp.minimum(hp, 0.0)) - 1.0)
    h_ref[...] = hp

    q = jnp.tanh(jnp.dot(hp, fc1w_ref[...],
                         preferred_element_type=jnp.float32)
                 + fc1b_ref[...][None, :])
    sp = jnp.dot(q, fc2w_ref[...], preferred_element_type=jnp.float32)
    ssum = jnp.sum(sp)

    @pl.when(t == 0)
    def _init():
        s_ref[...] = jnp.zeros((1, 1), jnp.float32)

    s_ref[...] += ssum


def _tc_b(off, hidden, tgt3, a1, attn2, fc1_w, fc1_b, fc2_w):
    return pl.pallas_call(
        _tcb_body,
        grid=(NB,),
        in_specs=[
            pl.BlockSpec(memory_space=pltpu.SMEM),
            pl.BlockSpec(memory_space=pl.ANY),
            pl.BlockSpec(memory_space=pl.ANY),
            pl.BlockSpec((W, A1P), lambda t: (t, 0)),
            pl.BlockSpec((H, D), lambda t: (0, 0)),
            pl.BlockSpec((H * D, AV), lambda t: (0, 0)),
            pl.BlockSpec((AV,), lambda t: (0,)),
            pl.BlockSpec((AV, 1), lambda t: (0, 0)),
        ],
        out_specs=[
            pl.BlockSpec((W, H * D), lambda t: (t, 0)),
            pl.BlockSpec((1, 1), lambda t: (0, 0)),
        ],
        out_shape=[
            jax.ShapeDtypeStruct((N_TARGET, H * D), jnp.float32),
            jax.ShapeDtypeStruct((1, 1), jnp.float32),
        ],
        scratch_shapes=[
            pltpu.VMEM((C, D), jnp.float32),
            pltpu.VMEM((C, 1), jnp.int32),
            pltpu.VMEM((W, H * D), jnp.float32),
            pltpu.VMEM((W, H), jnp.float32),
            pltpu.SemaphoreType.DMA,
            pltpu.SemaphoreType.DMA,
        ],
    )(off, hidden, tgt3, a1, attn2, fc1_w, fc1_b, fc2_w)


def _tcc_body(h0_ref, h1_ref, s0_ref, s1_ref, fcuw_ref, fcub_ref,
              hu_ref, lg_ref, beta_ref):
    sv = jnp.concatenate([s0_ref[...], s1_ref[...]], axis=0) \
        * (1.0 / N_TARGET)
    ex = jnp.exp(sv - jnp.max(sv))
    beta = ex / jnp.sum(ex)

    t = pl.program_id(0)

    @pl.when(t == 0)
    def _():
        beta_ref[...] = beta

    hu = beta[0:1, 0:1] * h0_ref[...] + beta[1:2, 0:1] * h1_ref[...]
    hu_ref[...] = hu
    lg_ref[...] = jnp.dot(hu, fcuw_ref[...],
                          preferred_element_type=jnp.float32) \
        + fcub_ref[...][None, :]


def _tc_c(h0, h1, s0, s1, fc_user_w, fc_user_b):
    return pl.pallas_call(
        _tcc_body,
        grid=(NB,),
        in_specs=[
            pl.BlockSpec((W, H * D), lambda t: (t, 0)),
            pl.BlockSpec((W, H * D), lambda t: (t, 0)),
            pl.BlockSpec((1, 1), lambda t: (0, 0)),
            pl.BlockSpec((1, 1), lambda t: (0, 0)),
            pl.BlockSpec((H * D, OUT_DIM), lambda t: (0, 0)),
            pl.BlockSpec((OUT_DIM,), lambda t: (0,)),
        ],
        out_specs=[
            pl.BlockSpec((W, H * D), lambda t: (t, 0)),
            pl.BlockSpec((W, OUT_DIM), lambda t: (t, 0)),
            pl.BlockSpec((2, 1), lambda t: (0, 0)),
        ],
        out_shape=[
            jax.ShapeDtypeStruct((N_TARGET, H * D), jnp.float32),
            jax.ShapeDtypeStruct((N_TARGET, OUT_DIM), jnp.float32),
            jax.ShapeDtypeStruct((2, 1), jnp.float32),
        ],
    )(h0, h1, s0, s1, fc_user_w, fc_user_b)


def kernel(features, topic, type_mask,
           edge_metapath_indices_0, edge_metapath_indices_1,
           edge_metapath_text_indices_0, edge_metapath_text_indices_1,
           target_idx_0, target_idx_1, node_list_0, node_list_1,
           attn1, attn2, fc1_w, fc1_b, fc2_w, fc_user_w, fc_user_b):
    del type_mask
    zpad = jnp.zeros((C,), jnp.int32)
    bnd = jnp.arange(NB + 1, dtype=jnp.int32) * W
    attn1p = jnp.pad(attn1, ((0, 0), (0, A1P - H)))
    fa1 = _tc_fa1(features, attn1p)

    idxTs = [edge_metapath_indices_0.T.astype(jnp.int32).reshape(-1),
             edge_metapath_indices_1.T.astype(jnp.int32).reshape(-1)]
    txts = [edge_metapath_text_indices_0.astype(jnp.int32),
            edge_metapath_text_indices_1.astype(jnp.int32)]
    nls = [node_list_0.astype(jnp.int32), node_list_1.astype(jnp.int32)]
    tgts = [target_idx_0.astype(jnp.int32), target_idx_1.astype(jnp.int32)]

    hs, ss = [], []
    for m in range(2):
        tgt3 = jnp.concatenate([tgts[m], zpad]).reshape(E_PAD, 1)
        off = jnp.searchsorted(tgts[m], bnd).astype(jnp.int32)
        hidden, a1 = _sc_gather(features, topic, idxTs[m], txts[m],
                                nls[m], fa1)
        h, s = _tc_b(off, hidden, tgt3, a1, attn2, fc1_w, fc1_b, fc2_w)
        hs.append(h)
        ss.append(s)

    h_user, logits, beta2 = _tc_c(hs[0], hs[1], ss[0], ss[1],
                                  fc_user_w, fc_user_b)
    return h_user, logits, beta2.reshape(2)


# speedup vs baseline: 2.0706x; 1.3521x over previous
import jax
import jax.numpy as jnp
from jax import lax
from jax.experimental import pallas as pl
from jax.experimental.pallas import tpu as pltpu
from jax.experimental.pallas import tpu_sc as plsc

N_NODES = 10000
N_TARGET = 8192
E = 160000
L = 3
D = 128
H = 4
AV = 128
OUT_DIM = 128

W = 256
NB = N_TARGET // W
C = 1024
E_PAD = E + C

NC = 2
NS = 16
NW = NC * NS
EDGES_PER_W = E // NW
EK = 200
ECH = EDGES_PER_W // EK
CEN_PER_W = N_TARGET // NW
CK = 128
A1P = 128


def _sc_gather_body(feat, topic, idxT, txt, nlst, fa1,
                    hid_out, a1_out,
                    ie0, ie1, ie2, ie3, r0, r1, r2, r3,
                    icen, rcen, sema, semc):
    wid = lax.axis_index("s") * NC + lax.axis_index("c")
    ebase = wid * EDGES_PER_W

    def fire(k):
        b = ebase + k * EK
        pltpu.sync_copy(idxT.at[pl.ds(b, EK)], ie0)
        pltpu.sync_copy(idxT.at[pl.ds(E + b, EK)], ie1)
        pltpu.sync_copy(idxT.at[pl.ds(2 * E + b, EK)], ie2)
        pltpu.sync_copy(txt.at[pl.ds(b, EK)], ie3)
        pltpu.async_copy(feat.at[ie0], r0, sema)
        pltpu.async_copy(feat.at[ie1], r1, sema)
        pltpu.async_copy(feat.at[ie2], r2, sema)
        pltpu.async_copy(topic.at[ie3], r3, sema)

    def drain():
        pltpu.make_async_copy(feat.at[ie0], r0, sema).wait()
        pltpu.make_async_copy(feat.at[ie1], r1, sema).wait()
        pltpu.make_async_copy(feat.at[ie2], r2, sema).wait()
        pltpu.make_async_copy(topic.at[ie3], r3, sema).wait()

    def compute_store(k):
        def erow(e, c2):
            for u in range(2):
                ee = e * 2 + u
                for d8 in range(D // 16):
                    s = pl.ds(d8 * 16, 16)
                    r0[ee, s] = (r0[ee, s] + r1[ee, s] + r2[ee, s]) \
                        * (1.0 / 3.0) + r3[ee, s]
            return c2
        lax.fori_loop(0, EK // 2, erow, 0)
        pltpu.sync_copy(r0, hid_out.at[pl.ds(ebase + k * EK, EK)])

    def body(k, carry):
        fire(k)
        drain()
        compute_store(k)
        return carry
    lax.fori_loop(0, ECH, body, 0)

    cbase = wid * CEN_PER_W

    def cchunk(k, carry):
        b = cbase + k * CK
        pltpu.sync_copy(nlst.at[pl.ds(b, CK)], icen)
        pltpu.async_copy(fa1.at[icen], rcen, semc).wait()
        pltpu.sync_copy(rcen, a1_out.at[pl.ds(b, CK)])
        return carry
    lax.fori_loop(0, CEN_PER_W // CK, cchunk, 0)


def _sc_gather(feat, topic, idxT, txt, nlst, fa1):
    mesh = plsc.VectorSubcoreMesh(core_axis_name="c", subcore_axis_name="s")
    fn = pl.kernel(
        _sc_gather_body,
        mesh=mesh,
        out_type=(
            jax.ShapeDtypeStruct((E_PAD, D), jnp.float32),
            jax.ShapeDtypeStruct((N_TARGET, A1P), jnp.float32),
        ),
        scratch_types=(
            [pltpu.VMEM((EK,), jnp.int32) for _ in range(4)]
            + [pltpu.VMEM((EK, D), jnp.float32) for _ in range(4)]
            + [pltpu.VMEM((CK,), jnp.int32),
               pltpu.VMEM((CK, A1P), jnp.float32),
               pltpu.SemaphoreType.DMA,
               pltpu.SemaphoreType.DMA]
        ),
    )
    return fn(feat, topic, idxT, txt, nlst, fa1)


def _fa1_body(feat_ref, attn1p_ref, out_ref):
    out_ref[...] = jnp.dot(feat_ref[...], attn1p_ref[...],
                           preferred_element_type=jnp.float32)


def _tc_fa1(features, attn1p):
    return pl.pallas_call(
        _fa1_body,
        out_shape=jax.ShapeDtypeStruct((N_NODES, A1P), jnp.float32),
    )(features, attn1p)


def _tcb_body(off_ref, hid_hbm, tgt_hbm, a1_ref, attn2_ref,
              fc1w_ref, fc1b_ref, fc2w_ref,
              h_ref, s_ref, hid_buf, tgt_buf, accn, accd, sem1, sem2):
    t = pl.program_id(0)
    start = off_ref[t]
    end = off_ref[t + 1]
    astart = (start // C) * C
    trip = (end - astart + C - 1) // C

    accn[...] = jnp.zeros((W, H * D), jnp.float32)
    accd[...] = jnp.zeros((W, H), jnp.float32)

    def dma_pair(c):
        p = c & 1
        s0 = astart + c * C
        cp1 = pltpu.make_async_copy(hid_hbm.at[pl.ds(s0, C), :],
                                    hid_buf.at[p], sem1.at[p])
        cp2 = pltpu.make_async_copy(tgt_hbm.at[pl.ds(s0, C), :],
                                    tgt_buf.at[p], sem2.at[p])
        return cp1, cp2

    @pl.when(trip > 0)
    def _prime():
        cp1, cp2 = dma_pair(0)
        cp1.start()
        cp2.start()

    def chunk(c, carry):
        @pl.when(c + 1 < trip)
        def _next():
            cp1, cp2 = dma_pair(c + 1)
            cp1.start()
            cp2.start()
        p = c & 1
        s0 = astart + c * C
        cp1, cp2 = dma_pair(c)
        cp1.wait()
        cp2.wait()
        tv = tgt_buf[p]
        jg = lax.broadcasted_iota(jnp.int32, (C, 1), 0) + s0
        mask = (jg >= start) & (jg < end)
        rel = jnp.clip(tv - t * W, 0, W - 1)
        onehot = jnp.where(
            (rel == lax.broadcasted_iota(jnp.int32, (C, W), 1)) & mask,
            1.0, 0.0)
        hidm = jnp.where(mask, hid_buf[p], 0.0)
        a2 = lax.dot_general(hidm, attn2_ref[...],
                             (((1,), (1,)), ((), ())),
                             preferred_element_type=jnp.float32)
        a1e = jnp.dot(onehot, a1_ref[:, :H],
                      preferred_element_type=jnp.float32)
        a = a1e + a2
        a = jnp.where(a > 0, a, 0.01 * a)
        ae = jnp.where(mask, jnp.exp(a), 0.0)
        vals = jnp.concatenate(
            [ae[:, h:h + 1] * hidm for h in range(H)], axis=1)
        accn[...] += lax.dot_general(onehot, vals,
                                     (((0,), (0,)), ((), ())),
                                     preferred_element_type=jnp.float32)
        accd[...] += lax.dot_general(onehot, ae,
                                     (((0,), (0,)), ((), ())),
                                     preferred_element_type=jnp.float32)
        return carry
    lax.fori_loop(0, trip, chunk, 0)

    num = accn[...]
    den = accd[...]
    hp = jnp.concatenate(
        [num[:, h * D:(h + 1) * D] / (den[:, h:h + 1] + 1e-9)
         for h in range(H)], axis=1)
    hp = jnp.where(hp > 0, hp, jnp.exp(jnp.minimum(hp, 0.0)) - 1.0)
    h_ref[...] = hp

    q = jnp.tanh(jnp.dot(hp, fc1w_ref[...],
                         preferred_element_type=jnp.float32)
                 + fc1b_ref[...][None, :])
    sp = jnp.dot(q, fc2w_ref[...], preferred_element_type=jnp.float32)
    ssum = jnp.sum(sp)

    @pl.when(t == 0)
    def _init():
        s_ref[...] = jnp.zeros((1, 1), jnp.float32)

    s_ref[...] += ssum


def _tc_b(off, hidden, tgt3, a1, attn2, fc1_w, fc1_b, fc2_w):
    return pl.pallas_call(
        _tcb_body,
        grid=(NB,),
        in_specs=[
            pl.BlockSpec(memory_space=pltpu.SMEM),
            pl.BlockSpec(memory_space=pl.ANY),
            pl.BlockSpec(memory_space=pl.ANY),
            pl.BlockSpec((W, A1P), lambda t: (t, 0)),
            pl.BlockSpec((H, D), lambda t: (0, 0)),
            pl.BlockSpec((H * D, AV), lambda t: (0, 0)),
            pl.BlockSpec((AV,), lambda t: (0,)),
            pl.BlockSpec((AV, 1), lambda t: (0, 0)),
        ],
        out_specs=[
            pl.BlockSpec((W, H * D), lambda t: (t, 0)),
            pl.BlockSpec((1, 1), lambda t: (0, 0)),
        ],
        out_shape=[
            jax.ShapeDtypeStruct((N_TARGET, H * D), jnp.float32),
            jax.ShapeDtypeStruct((1, 1), jnp.float32),
        ],
        scratch_shapes=[
            pltpu.VMEM((2, C, D), jnp.float32),
            pltpu.VMEM((2, C, 1), jnp.int32),
            pltpu.VMEM((W, H * D), jnp.float32),
            pltpu.VMEM((W, H), jnp.float32),
            pltpu.SemaphoreType.DMA((2,)),
            pltpu.SemaphoreType.DMA((2,)),
        ],
    )(off, hidden, tgt3, a1, attn2, fc1_w, fc1_b, fc2_w)


def _tcc_body(h0_ref, h1_ref, s0_ref, s1_ref, fcuw_ref, fcub_ref,
              hu_ref, lg_ref, beta_ref):
    sv = jnp.concatenate([s0_ref[...], s1_ref[...]], axis=0) \
        * (1.0 / N_TARGET)
    ex = jnp.exp(sv - jnp.max(sv))
    beta = ex / jnp.sum(ex)

    t = pl.program_id(0)

    @pl.when(t == 0)
    def _():
        beta_ref[...] = beta

    hu = beta[0:1, 0:1] * h0_ref[...] + beta[1:2, 0:1] * h1_ref[...]
    hu_ref[...] = hu
    lg_ref[...] = jnp.dot(hu, fcuw_ref[...],
                          preferred_element_type=jnp.float32) \
        + fcub_ref[...][None, :]


def _tc_c(h0, h1, s0, s1, fc_user_w, fc_user_b):
    return pl.pallas_call(
        _tcc_body,
        grid=(NB,),
        in_specs=[
            pl.BlockSpec((W, H * D), lambda t: (t, 0)),
            pl.BlockSpec((W, H * D), lambda t: (t, 0)),
            pl.BlockSpec((1, 1), lambda t: (0, 0)),
            pl.BlockSpec((1, 1), lambda t: (0, 0)),
            pl.BlockSpec((H * D, OUT_DIM), lambda t: (0, 0)),
            pl.BlockSpec((OUT_DIM,), lambda t: (0,)),
        ],
        out_specs=[
            pl.BlockSpec((W, H * D), lambda t: (t, 0)),
            pl.BlockSpec((W, OUT_DIM), lambda t: (t, 0)),
            pl.BlockSpec((2, 1), lambda t: (0, 0)),
        ],
        out_shape=[
            jax.ShapeDtypeStruct((N_TARGET, H * D), jnp.float32),
            jax.ShapeDtypeStruct((N_TARGET, OUT_DIM), jnp.float32),
            jax.ShapeDtypeStruct((2, 1), jnp.float32),
        ],
    )(h0, h1, s0, s1, fc_user_w, fc_user_b)


def kernel(features, topic, type_mask,
           edge_metapath_indices_0, edge_metapath_indices_1,
           edge_metapath_text_indices_0, edge_metapath_text_indices_1,
           target_idx_0, target_idx_1, node_list_0, node_list_1,
           attn1, attn2, fc1_w, fc1_b, fc2_w, fc_user_w, fc_user_b):
    del type_mask
    zpad = jnp.zeros((C,), jnp.int32)
    bnd = jnp.arange(NB + 1, dtype=jnp.int32) * W
    attn1p = jnp.pad(attn1, ((0, 0), (0, A1P - H)))
    fa1 = _tc_fa1(features, attn1p)

    idxTs = [edge_metapath_indices_0.T.astype(jnp.int32).reshape(-1),
             edge_metapath_indices_1.T.astype(jnp.int32).reshape(-1)]
    txts = [edge_metapath_text_indices_0.astype(jnp.int32),
            edge_metapath_text_indices_1.astype(jnp.int32)]
    nls = [node_list_0.astype(jnp.int32), node_list_1.astype(jnp.int32)]
    tgts = [target_idx_0.astype(jnp.int32), target_idx_1.astype(jnp.int32)]

    hs, ss = [], []
    for m in range(2):
        tgt3 = jnp.concatenate([tgts[m], zpad]).reshape(E_PAD, 1)
        off = jnp.searchsorted(tgts[m], bnd).astype(jnp.int32)
        hidden, a1 = _sc_gather(features, topic, idxTs[m], txts[m],
                                nls[m], fa1)
        h, s = _tc_b(off, hidden, tgt3, a1, attn2, fc1_w, fc1_b, fc2_w)
        hs.append(h)
        ss.append(s)

    h_user, logits, beta2 = _tc_c(hs[0], hs[1], ss[0], ss[1],
                                  fc_user_w, fc_user_b)
    return h_user, logits, beta2.reshape(2)
